# Initial kernel scaffold; baseline (speedup 1.0000x reference)
import jax
import jax.numpy as jnp
from jax.experimental import pallas as pl


def kernel(tokens, W):
    raise NotImplementedError("write your pallas kernel here")


# speedup vs baseline: 1.2710x; 1.2710x over previous
import jax
import jax.numpy as jnp
from jax.experimental import pallas as pl

_COMMITMENT_COST = 0.25
_TS = 1024


def _body(x_ref, rs_ref, ws_ref, w_ref, qst_ref, idx_ref, ls_ref):
    x = x_ref[...]
    w = w_ref[...]
    dot = jax.lax.dot_general(
        x, w, (((1,), (1,)), ((), ())), preferred_element_type=jnp.float32
    )
    dist = (rs_ref[...] - 2.0 * dot) + ws_ref[...]
    md = jnp.min(dist, axis=1, keepdims=True)
    cidx = jax.lax.broadcasted_iota(jnp.int32, dist.shape, 1)
    ncodes = dist.shape[1]
    idx = jnp.min(jnp.where(dist == md, cidx, ncodes), axis=1)
    onehot = (cidx == idx[:, None]).astype(jnp.float32)
    q = jax.lax.dot_general(
        onehot, w, (((1,), (0,)), ((), ())), preferred_element_type=jnp.float32
    )
    qst_ref[...] = x + (q - x)
    idx_ref[...] = idx.reshape(1, 1, _TS)
    ls_ref[...] = jnp.full((1, 1, 128), jnp.sum((q - x) ** 2), jnp.float32)


def kernel(tokens, W):
    B, K, D = tokens.shape
    C = W.shape[0]
    N = B * K
    G = N // _TS
    flat = tokens.reshape(N, D)
    rowsq = jnp.sum(flat ** 2, axis=1, keepdims=True)
    wsq = jnp.sum(W ** 2, axis=1).reshape(1, C)

    qst, idx3, ls3 = pl.pallas_call(
        _body,
        grid=(G,),
        in_specs=[
            pl.BlockSpec((_TS, D), lambda i: (i, 0)),
            pl.BlockSpec((_TS, 1), lambda i: (i, 0)),
            pl.BlockSpec((1, C), lambda i: (0, 0)),
            pl.BlockSpec((C, D), lambda i: (0, 0)),
        ],
        out_specs=[
            pl.BlockSpec((_TS, D), lambda i: (i, 0)),
            pl.BlockSpec((1, 1, _TS), lambda i: (i, 0, 0)),
            pl.BlockSpec((1, 1, 128), lambda i: (i, 0, 0)),
        ],
        out_shape=[
            jax.ShapeDtypeStruct((N, D), jnp.float32),
            jax.ShapeDtypeStruct((G, 1, _TS), jnp.int32),
            jax.ShapeDtypeStruct((G, 1, 128), jnp.float32),
        ],
    )(flat, rowsq, wsq, W)

    m = jnp.sum(ls3[:, 0, 0]) / (N * D)
    vq_loss = _COMMITMENT_COST * m + m
    return qst.reshape(B, K, D), vq_loss, idx3.reshape(B, K)


# speedup vs baseline: 1.5207x; 1.1965x over previous
import jax
import jax.numpy as jnp
from jax.experimental import pallas as pl

_COMMITMENT_COST = 0.25
_TS = 1024


def _body(x_ref, rs_ref, ws_ref, w2_ref, wsplit_ref, qst_ref, idx_ref, ls_ref):
    x = x_ref[...]
    w2 = w2_ref[...]
    dot2t = jax.lax.dot_general(
        w2, x, (((1,), (1,)), ((), ())), preferred_element_type=jnp.float32
    )
    dist = (rs_ref[0] - dot2t) + ws_ref[...]
    md = jnp.min(dist, axis=0, keepdims=True)
    cidxf = jax.lax.broadcasted_iota(jnp.int32, dist.shape, 0).astype(jnp.float32)
    ncodes = float(dist.shape[0])
    idxf = jnp.min(jnp.where(dist == md, cidxf, ncodes), axis=0, keepdims=True)
    idx_ref[...] = idxf.astype(jnp.int32).reshape(1, 1, _TS)
    onehot = (cidxf == idxf).astype(jnp.bfloat16)
    qcat = jax.lax.dot_general(
        onehot, wsplit_ref[...], (((0,), (0,)), ((), ())),
        preferred_element_type=jnp.float32,
    )
    D = x.shape[1]
    q = (qcat[:, :D] + qcat[:, D:2 * D]) + qcat[:, 2 * D:]
    qst_ref[...] = x + (q - x)
    ls_ref[...] = jnp.full((1, 1, 128), jnp.sum((q - x) ** 2), jnp.float32)


def kernel(tokens, W):
    B, K, D = tokens.shape
    C = W.shape[0]
    N = B * K
    G = N // _TS
    flat = tokens.reshape(N, D)
    rowsq = jnp.sum(flat ** 2, axis=1, keepdims=True)
    rowsq3 = rowsq.reshape(G, 1, _TS)
    wsq = jnp.sum(W ** 2, axis=1).reshape(C, 1)
    W2 = W * 2.0
    wbits = W.view(jnp.int32)
    hi = (wbits & jnp.int32(-65536)).view(jnp.float32)
    rem = W - hi
    mid = (rem.view(jnp.int32) & jnp.int32(-65536)).view(jnp.float32)
    lo = rem - mid
    wsplit = jnp.concatenate(
        [hi.astype(jnp.bfloat16), mid.astype(jnp.bfloat16),
         lo.astype(jnp.bfloat16)], axis=1)

    qst, idx3, ls3 = pl.pallas_call(
        _body,
        grid=(G,),
        in_specs=[
            pl.BlockSpec((_TS, D), lambda i: (i, 0)),
            pl.BlockSpec((1, 1, _TS), lambda i: (i, 0, 0)),
            pl.BlockSpec((C, 1), lambda i: (0, 0)),
            pl.BlockSpec((C, D), lambda i: (0, 0)),
            pl.BlockSpec((C, 3 * D), lambda i: (0, 0)),
        ],
        out_specs=[
            pl.BlockSpec((_TS, D), lambda i: (i, 0)),
            pl.BlockSpec((1, 1, _TS), lambda i: (i, 0, 0)),
            pl.BlockSpec((1, 1, 128), lambda i: (i, 0, 0)),
        ],
        out_shape=[
            jax.ShapeDtypeStruct((N, D), jnp.float32),
            jax.ShapeDtypeStruct((G, 1, _TS), jnp.int32),
            jax.ShapeDtypeStruct((G, 1, 128), jnp.float32),
        ],
    )(flat, rowsq3, wsq, W2, wsplit)

    m = jnp.sum(ls3[:, 0, 0]) / (N * D)
    vq_loss = _COMMITMENT_COST * m + m
    return qst.reshape(B, K, D), vq_loss, idx3.reshape(B, K)


# speedup vs baseline: 1.8295x; 1.2031x over previous
import jax
import jax.numpy as jnp
from jax.experimental import pallas as pl

_COMMITMENT_COST = 0.25
_TS = 2048


def _body(x_ref, rs_ref, ws_ref, w2_ref, wsplit_ref, g_ref, qst_ref, idx_ref,
          ls_ref):
    x = x_ref[...]
    w2 = w2_ref[...]
    dot2t = jax.lax.dot_general(
        w2, x, (((1,), (1,)), ((), ())), preferred_element_type=jnp.float32
    )
    dist = (rs_ref[0] - dot2t) + ws_ref[...]
    md = jnp.min(dist, axis=0, keepdims=True)
    maskbf = (dist == md).astype(jnp.bfloat16)
    s = jax.lax.dot_general(
        g_ref[...], maskbf, (((0,), (0,)), ((), ())),
        preferred_element_type=jnp.float32,
    )
    e = jax.lax.bitcast_convert_type(s, jnp.int32) >> 23
    giota = jax.lax.broadcasted_iota(jnp.int32, s.shape, 0)
    cand = (16 * giota + 127 - e).astype(jnp.float32)
    cand = jnp.where(s > 0.0, cand, float(dist.shape[0]))
    idxf = jnp.min(cand, axis=0, keepdims=True)
    idx_ref[...] = idxf.astype(jnp.int32).reshape(1, 1, _TS)
    cidxf = jax.lax.broadcasted_iota(jnp.int32, dist.shape, 0).astype(jnp.float32)
    onehot = (cidxf == idxf).astype(jnp.bfloat16)
    qcat = jax.lax.dot_general(
        onehot, wsplit_ref[...], (((0,), (0,)), ((), ())),
        preferred_element_type=jnp.float32,
    )
    D = x.shape[1]
    q = (qcat[:, :D] + qcat[:, D:2 * D]) + qcat[:, 2 * D:]
    qst_ref[...] = x + (q - x)
    ls_ref[...] = jnp.full((1, 1, 128), jnp.sum((q - x) ** 2), jnp.float32)


def kernel(tokens, W):
    B, K, D = tokens.shape
    C = W.shape[0]
    N = B * K
    G = N // _TS
    flat = tokens.reshape(N, D)
    rowsq = jnp.sum(flat ** 2, axis=1, keepdims=True)
    rowsq3 = rowsq.reshape(G, 1, _TS)
    wsq = jnp.sum(W ** 2, axis=1).reshape(C, 1)
    W2 = W * 2.0
    wbits = W.view(jnp.int32)
    hi = (wbits & jnp.int32(-65536)).view(jnp.float32)
    rem = W - hi
    mid = (rem.view(jnp.int32) & jnp.int32(-65536)).view(jnp.float32)
    lo = rem - mid
    wsplit = jnp.concatenate(
        [hi.astype(jnp.bfloat16), mid.astype(jnp.bfloat16),
         lo.astype(jnp.bfloat16)], axis=1)
    carange = jnp.arange(C)
    gmat = jnp.where(
        (carange[:, None] // 16) == jnp.arange(C // 16)[None, :],
        2.0 ** (-(carange[:, None] % 16)).astype(jnp.float32),
        0.0,
    ).astype(jnp.bfloat16)

    qst, idx3, ls3 = pl.pallas_call(
        _body,
        grid=(G,),
        in_specs=[
            pl.BlockSpec((_TS, D), lambda i: (i, 0)),
            pl.BlockSpec((1, 1, _TS), lambda i: (i, 0, 0)),
            pl.BlockSpec((C, 1), lambda i: (0, 0)),
            pl.BlockSpec((C, D), lambda i: (0, 0)),
            pl.BlockSpec((C, 3 * D), lambda i: (0, 0)),
            pl.BlockSpec((C, C // 16), lambda i: (0, 0)),
        ],
        out_specs=[
            pl.BlockSpec((_TS, D), lambda i: (i, 0)),
            pl.BlockSpec((1, 1, _TS), lambda i: (i, 0, 0)),
            pl.BlockSpec((1, 1, 128), lambda i: (i, 0, 0)),
        ],
        out_shape=[
            jax.ShapeDtypeStruct((N, D), jnp.float32),
            jax.ShapeDtypeStruct((G, 1, _TS), jnp.int32),
            jax.ShapeDtypeStruct((G, 1, 128), jnp.float32),
        ],
    )(flat, rowsq3, wsq, W2, wsplit, gmat)

    m = jnp.sum(ls3[:, 0, 0]) / (N * D)
    vq_loss = _COMMITMENT_COST * m + m
    return qst.reshape(B, K, D), vq_loss, idx3.reshape(B, K)


# speedup vs baseline: 1.8808x; 1.0280x over previous
import jax
import jax.numpy as jnp
from jax.experimental import pallas as pl

_COMMITMENT_COST = 0.25
_TS = 4096


def _body(x_ref, rs_ref, ws_ref, w2_ref, wsplit_ref, g_ref, qst_ref, idx_ref,
          ls_ref):
    x = x_ref[...]
    w2 = w2_ref[...]
    dot2t = jax.lax.dot_general(
        w2, x, (((1,), (1,)), ((), ())), preferred_element_type=jnp.float32
    )
    dist = (rs_ref[0] - dot2t) + ws_ref[...]
    md = jnp.min(dist, axis=0, keepdims=True)
    maskbf = (dist == md).astype(jnp.bfloat16)
    s = jax.lax.dot_general(
        g_ref[...], maskbf, (((0,), (0,)), ((), ())),
        preferred_element_type=jnp.float32,
    )
    e = jax.lax.bitcast_convert_type(s, jnp.int32) >> 23
    giota = jax.lax.broadcasted_iota(jnp.int32, s.shape, 0)
    cand = (16 * giota + 127 - e).astype(jnp.float32)
    cand = jnp.where(s > 0.0, cand, float(dist.shape[0]))
    idxf = jnp.min(cand, axis=0, keepdims=True)
    idxi = idxf.astype(jnp.int32)
    idx_ref[...] = idxi.reshape(1, 1, _TS)
    cidx = jax.lax.broadcasted_iota(jnp.int32, dist.shape, 0)
    onehot = (cidx == idxi).astype(jnp.bfloat16)
    qcat = jax.lax.dot_general(
        onehot, wsplit_ref[...], (((0,), (0,)), ((), ())),
        preferred_element_type=jnp.float32,
    )
    D = x.shape[1]
    q = (qcat[:, :D] + qcat[:, D:2 * D]) + qcat[:, 2 * D:]
    qst_ref[...] = x + (q - x)
    ls_ref[...] = jnp.full((1, 1, 128), jnp.sum((q - x) ** 2), jnp.float32)


def kernel(tokens, W):
    B, K, D = tokens.shape
    C = W.shape[0]
    N = B * K
    G = N // _TS
    flat = tokens.reshape(N, D)
    rowsq = jnp.sum(flat ** 2, axis=1, keepdims=True)
    rowsq3 = rowsq.reshape(G, 1, _TS)
    wsq = jnp.sum(W ** 2, axis=1).reshape(C, 1)
    W2 = W * 2.0
    wbits = W.view(jnp.int32)
    hi = (wbits & jnp.int32(-65536)).view(jnp.float32)
    rem = W - hi
    mid = (rem.view(jnp.int32) & jnp.int32(-65536)).view(jnp.float32)
    lo = rem - mid
    wsplit = jnp.concatenate(
        [hi.astype(jnp.bfloat16), mid.astype(jnp.bfloat16),
         lo.astype(jnp.bfloat16)], axis=1)
    carange = jnp.arange(C)
    gmat = jnp.where(
        (carange[:, None] // 16) == jnp.arange(C // 16)[None, :],
        2.0 ** (-(carange[:, None] % 16)).astype(jnp.float32),
        0.0,
    ).astype(jnp.bfloat16)

    qst, idx3, ls3 = pl.pallas_call(
        _body,
        grid=(G,),
        in_specs=[
            pl.BlockSpec((_TS, D), lambda i: (i, 0)),
            pl.BlockSpec((1, 1, _TS), lambda i: (i, 0, 0)),
            pl.BlockSpec((C, 1), lambda i: (0, 0)),
            pl.BlockSpec((C, D), lambda i: (0, 0)),
            pl.BlockSpec((C, 3 * D), lambda i: (0, 0)),
            pl.BlockSpec((C, C // 16), lambda i: (0, 0)),
        ],
        out_specs=[
            pl.BlockSpec((_TS, D), lambda i: (i, 0)),
            pl.BlockSpec((1, 1, _TS), lambda i: (i, 0, 0)),
            pl.BlockSpec((1, 1, 128), lambda i: (i, 0, 0)),
        ],
        out_shape=[
            jax.ShapeDtypeStruct((N, D), jnp.float32),
            jax.ShapeDtypeStruct((G, 1, _TS), jnp.int32),
            jax.ShapeDtypeStruct((G, 1, 128), jnp.float32),
        ],
    )(flat, rowsq3, wsq, W2, wsplit, gmat)

    m = jnp.sum(ls3[:, 0, 0]) / (N * D)
    vq_loss = _COMMITMENT_COST * m + m
    return qst.reshape(B, K, D), vq_loss, idx3.reshape(B, K)
